# Initial kernel scaffold; baseline (speedup 1.0000x reference)
import jax
import jax.numpy as jnp
from jax.experimental import pallas as pl


def kernel(pos, batch):
    raise NotImplementedError("write your pallas kernel here")


# speedup vs baseline: 5.3966x; 5.3966x over previous
import functools

import jax
import jax.numpy as jnp
from jax import lax
from jax.experimental import pallas as pl
from jax.experimental.pallas import tpu as pltpu
from jax.experimental.pallas import tpu_sc as plsc

_CUTOFF2 = 25.0
_K = 32
_TM = 128

_SC_CORES = 2
_SC_SUBCORES = 16
_NW = _SC_CORES * _SC_SUBCORES


def _topk_body(pos_r, pos_c, bat_r, bat_c, src_ref, wgt_ref, masked_ref):
    i = pl.program_id(0)
    tm = pos_r.shape[0]
    n = pos_c.shape[1]

    xr = pos_r[:, 0:1]
    yr = pos_r[:, 1:2]
    zr = pos_r[:, 2:3]
    xc = pos_c[0:1, :]
    yc = pos_c[1:2, :]
    zc = pos_c[2:3, :]

    dx = xr - xc
    dy = yr - yc
    dz = zr - zc
    d2 = dx * dx + dy * dy + dz * dz

    same = bat_r[...] == bat_c[...]
    valid = same & (d2 <= _CUTOFF2)
    inf = jnp.float32(jnp.inf)
    masked_ref[...] = jnp.where(valid, d2, inf)

    iota_j = lax.broadcasted_iota(jnp.int32, (tm, n), 1)
    rowid = i * tm + lax.broadcasted_iota(jnp.int32, (tm, 1), 0)
    slot = lax.broadcasted_iota(jnp.int32, (tm, _K), 1)

    def body(t, carry):
        src_acc, wgt_acc = carry
        m = masked_ref[...]
        mvals = jnp.min(m, axis=1, keepdims=True)
        eq = m == mvals
        jcand = jnp.where(eq, iota_j, n)
        jmin = jnp.min(jcand, axis=1, keepdims=True)
        masked_ref[...] = jnp.where(iota_j == jmin, inf, m)
        finite = mvals < inf
        srcv = jnp.where(finite, jmin, rowid)
        loop_m = srcv != rowid
        safe = jnp.sqrt(jnp.where(loop_m, mvals, 1.0))
        w = jnp.where(loop_m, safe, 0.0)
        sel = slot == t
        src_acc = jnp.where(sel, srcv, src_acc)
        wgt_acc = jnp.where(sel, w, wgt_acc)
        return src_acc, wgt_acc

    src0 = jnp.zeros((tm, _K), jnp.int32)
    wgt0 = jnp.zeros((tm, _K), jnp.float32)
    src, wgt = lax.fori_loop(0, _K, body, (src0, wgt0))
    src_ref[...] = src
    wgt_ref[...] = wgt


def _topk_call(pos, batch, interpret=False):
    n = pos.shape[0]
    grid = n // _TM
    pos_c = pos.T
    bat_r = batch.reshape(n, 1)
    bat_c = batch.reshape(1, n)
    return pl.pallas_call(
        _topk_body,
        grid=(grid,),
        in_specs=[
            pl.BlockSpec((_TM, 3), lambda i: (i, 0)),
            pl.BlockSpec((3, n), lambda i: (0, 0)),
            pl.BlockSpec((_TM, 1), lambda i: (i, 0)),
            pl.BlockSpec((1, n), lambda i: (0, 0)),
        ],
        out_specs=[
            pl.BlockSpec((_TM, _K), lambda i: (i, 0)),
            pl.BlockSpec((_TM, _K), lambda i: (i, 0)),
        ],
        out_shape=[
            jax.ShapeDtypeStruct((n, _K), jnp.int32),
            jax.ShapeDtypeStruct((n, _K), jnp.float32),
        ],
        scratch_shapes=[pltpu.VMEM((_TM, n), jnp.float32)],
        interpret=interpret,
    )(pos, pos_c, bat_r, bat_c)


def _edge_vec_call(px, py, pz, src_flat):
    n = px.shape[0]
    e = src_flat.shape[0]
    epw = e // _NW
    mesh = plsc.VectorSubcoreMesh(
        core_axis_name="c", subcore_axis_name="s",
        num_cores=_SC_CORES, num_subcores=_SC_SUBCORES)

    @functools.partial(
        pl.kernel,
        mesh=mesh,
        compiler_params=pltpu.CompilerParams(needs_layout_passes=False),
        out_type=[jax.ShapeDtypeStruct((e,), jnp.float32)] * 3,
        scratch_types=[
            pltpu.VMEM((n,), jnp.float32),
            pltpu.VMEM((n,), jnp.float32),
            pltpu.VMEM((n,), jnp.float32),
            pltpu.VMEM((epw,), jnp.int32),
            pltpu.VMEM((epw,), jnp.float32),
            pltpu.VMEM((epw,), jnp.float32),
            pltpu.VMEM((epw,), jnp.float32),
        ],
    )
    def k(px_hbm, py_hbm, pz_hbm, src_hbm, vx_hbm, vy_hbm, vz_hbm,
          px_v, py_v, pz_v, src_v, vx_v, vy_v, vz_v):
        c = lax.axis_index("c")
        s = lax.axis_index("s")
        wid = s * _SC_CORES + c
        base = wid * epw
        pltpu.sync_copy(px_hbm, px_v)
        pltpu.sync_copy(py_hbm, py_v)
        pltpu.sync_copy(pz_hbm, pz_v)
        pltpu.sync_copy(src_hbm.at[pl.ds(base, epw)], src_v)
        lane = lax.iota(jnp.int32, 16)

        def body(t, _):
            off = t * 16
            j = src_v[pl.ds(off, 16)]
            i = lax.shift_right_logical(base + off + lane, 5)
            vx_v[pl.ds(off, 16)] = (plsc.load_gather(px_v, [j])
                                    - plsc.load_gather(px_v, [i]))
            vy_v[pl.ds(off, 16)] = (plsc.load_gather(py_v, [j])
                                    - plsc.load_gather(py_v, [i]))
            vz_v[pl.ds(off, 16)] = (plsc.load_gather(pz_v, [j])
                                    - plsc.load_gather(pz_v, [i]))
            return 0

        lax.fori_loop(0, epw // 16, body, 0)
        pltpu.sync_copy(vx_v, vx_hbm.at[pl.ds(base, epw)])
        pltpu.sync_copy(vy_v, vy_hbm.at[pl.ds(base, epw)])
        pltpu.sync_copy(vz_v, vz_hbm.at[pl.ds(base, epw)])

    return k(px, py, pz, src_flat)


def kernel(pos, batch):
    n = pos.shape[0]
    src2d, wgt2d = _topk_call(pos, batch)
    src_flat = src2d.reshape(-1)
    px = pos[:, 0]
    py = pos[:, 1]
    pz = pos[:, 2]
    vx, vy, vz = _edge_vec_call(px, py, pz, src_flat)
    edge_vec = jnp.stack([vx, vy, vz], axis=-1)
    dst = jnp.broadcast_to(
        jnp.arange(n, dtype=jnp.int32)[:, None], (n, _K)).reshape(-1)
    edge_index = jnp.stack([src_flat, dst], axis=0)
    return edge_index, wgt2d.reshape(-1), edge_vec


# speedup vs baseline: 14.6593x; 2.7164x over previous
import functools

import jax
import jax.numpy as jnp
from jax import lax
from jax.experimental import pallas as pl
from jax.experimental.pallas import tpu as pltpu
from jax.experimental.pallas import tpu_sc as plsc

_CUTOFF2 = 25.0
_K = 32
_TM = 128

_SC_CORES = 2
_SC_SUBCORES = 16
_NW = _SC_CORES * _SC_SUBCORES


_CH = 512


def _topk_body(pos_r, pos_c, bat_r, bat_c, src_ref, wgt_ref, masked_ref):
    i = pl.program_id(0)
    tm = pos_r.shape[0]
    n = pos_c.shape[1]
    inf = jnp.float32(jnp.inf)

    br = bat_r[...]
    bat_full = bat_c[...]
    b0 = jnp.min(br)
    b1 = jnp.max(br)
    c0 = jnp.sum((bat_full < b0).astype(jnp.int32))
    c1 = jnp.sum((bat_full <= b1).astype(jnp.int32))
    ch0 = c0 // _CH
    ch1 = (c1 + _CH - 1) // _CH

    xr = pos_r[:, 0:1]
    yr = pos_r[:, 1:2]
    zr = pos_r[:, 2:3]

    def chunk_init(ch, _):
        sl = pl.ds(ch * _CH, _CH)
        dx = xr - pos_c[0:1, sl]
        dy = yr - pos_c[1:2, sl]
        dz = zr - pos_c[2:3, sl]
        d2 = dx * dx + dy * dy + dz * dz
        valid = (br == bat_c[0:1, sl]) & (d2 <= _CUTOFF2)
        masked_ref[:, sl] = jnp.where(valid, d2, inf)
        return 0

    lax.fori_loop(ch0, ch1, chunk_init, 0)

    iota_ch = lax.broadcasted_iota(jnp.int32, (tm, _CH), 1)
    rowid = i * tm + lax.broadcasted_iota(jnp.int32, (tm, 1), 0)
    slot = lax.broadcasted_iota(jnp.int32, (tm, _K), 1)

    def pass_body(carry):
        t, _cont, vprev, jprev, src_acc, wgt_acc = carry

        def scan_chunk(ch, sc):
            m, jm = sc
            sl = pl.ds(ch * _CH, _CH)
            c = masked_ref[:, sl]
            jj = iota_ch + ch * _CH
            elig = (c > vprev) | ((c == vprev) & (jj > jprev))
            ceff = jnp.where(elig, c, inf)
            v = jnp.min(ceff, axis=1, keepdims=True)
            jc = jnp.min(jnp.where(ceff == v, jj, n), axis=1, keepdims=True)
            upd = v < m
            return jnp.where(upd, v, m), jnp.where(upd, jc, jm)

        v, j = lax.fori_loop(
            ch0, ch1, scan_chunk,
            (jnp.full((tm, 1), inf, jnp.float32),
             jnp.full((tm, 1), n, jnp.int32)))
        finite = v < inf
        srcv = jnp.where(finite, j, rowid)
        loop_m = srcv != rowid
        safe = jnp.sqrt(jnp.where(loop_m, v, 1.0))
        w = jnp.where(loop_m, safe, 0.0)
        sel = slot == t
        src_acc = jnp.where(sel, srcv, src_acc)
        wgt_acc = jnp.where(sel, w, wgt_acc)
        cont = jnp.min(v) < inf
        return t + 1, cont, v, j, src_acc, wgt_acc

    def cond(carry):
        t, cont = carry[0], carry[1]
        return (t < _K) & cont

    src0 = jnp.broadcast_to(rowid, (tm, _K)).astype(jnp.int32)
    wgt0 = jnp.zeros((tm, _K), jnp.float32)
    carry0 = (jnp.int32(0), jnp.bool_(True),
              jnp.full((tm, 1), -jnp.inf, jnp.float32),
              jnp.full((tm, 1), -1, jnp.int32), src0, wgt0)
    out = lax.while_loop(cond, pass_body, carry0)
    src_ref[...] = out[4]
    wgt_ref[...] = out[5]


def _topk_call(pos, batch, interpret=False):
    n = pos.shape[0]
    grid = n // _TM
    pos_c = pos.T
    bat_r = batch.reshape(n, 1)
    bat_c = batch.reshape(1, n)
    return pl.pallas_call(
        _topk_body,
        grid=(grid,),
        in_specs=[
            pl.BlockSpec((_TM, 3), lambda i: (i, 0)),
            pl.BlockSpec((3, n), lambda i: (0, 0)),
            pl.BlockSpec((_TM, 1), lambda i: (i, 0)),
            pl.BlockSpec((1, n), lambda i: (0, 0)),
        ],
        out_specs=[
            pl.BlockSpec((_TM, _K), lambda i: (i, 0)),
            pl.BlockSpec((_TM, _K), lambda i: (i, 0)),
        ],
        out_shape=[
            jax.ShapeDtypeStruct((n, _K), jnp.int32),
            jax.ShapeDtypeStruct((n, _K), jnp.float32),
        ],
        scratch_shapes=[pltpu.VMEM((_TM, n), jnp.float32)],
        interpret=interpret,
    )(pos, pos_c, bat_r, bat_c)


def _edge_vec_call(px, py, pz, src_flat):
    n = px.shape[0]
    e = src_flat.shape[0]
    epw = e // _NW
    mesh = plsc.VectorSubcoreMesh(
        core_axis_name="c", subcore_axis_name="s",
        num_cores=_SC_CORES, num_subcores=_SC_SUBCORES)

    @functools.partial(
        pl.kernel,
        mesh=mesh,
        compiler_params=pltpu.CompilerParams(needs_layout_passes=False),
        out_type=[jax.ShapeDtypeStruct((e,), jnp.float32)] * 3,
        scratch_types=[
            pltpu.VMEM((n,), jnp.float32),
            pltpu.VMEM((n,), jnp.float32),
            pltpu.VMEM((n,), jnp.float32),
            pltpu.VMEM((epw,), jnp.int32),
            pltpu.VMEM((epw,), jnp.float32),
            pltpu.VMEM((epw,), jnp.float32),
            pltpu.VMEM((epw,), jnp.float32),
        ],
    )
    def k(px_hbm, py_hbm, pz_hbm, src_hbm, vx_hbm, vy_hbm, vz_hbm,
          px_v, py_v, pz_v, src_v, vx_v, vy_v, vz_v):
        c = lax.axis_index("c")
        s = lax.axis_index("s")
        wid = s * _SC_CORES + c
        base = wid * epw
        pltpu.sync_copy(px_hbm, px_v)
        pltpu.sync_copy(py_hbm, py_v)
        pltpu.sync_copy(pz_hbm, pz_v)
        pltpu.sync_copy(src_hbm.at[pl.ds(base, epw)], src_v)
        lane = lax.iota(jnp.int32, 16)

        def body(t, _):
            off = t * 16
            j = src_v[pl.ds(off, 16)]
            i = lax.shift_right_logical(base + off + lane, 5)
            vx_v[pl.ds(off, 16)] = (plsc.load_gather(px_v, [j])
                                    - plsc.load_gather(px_v, [i]))
            vy_v[pl.ds(off, 16)] = (plsc.load_gather(py_v, [j])
                                    - plsc.load_gather(py_v, [i]))
            vz_v[pl.ds(off, 16)] = (plsc.load_gather(pz_v, [j])
                                    - plsc.load_gather(pz_v, [i]))
            return 0

        lax.fori_loop(0, epw // 16, body, 0)
        pltpu.sync_copy(vx_v, vx_hbm.at[pl.ds(base, epw)])
        pltpu.sync_copy(vy_v, vy_hbm.at[pl.ds(base, epw)])
        pltpu.sync_copy(vz_v, vz_hbm.at[pl.ds(base, epw)])

    return k(px, py, pz, src_flat)


def kernel(pos, batch):
    n = pos.shape[0]
    src2d, wgt2d = _topk_call(pos, batch)
    src_flat = src2d.reshape(-1)
    px = pos[:, 0]
    py = pos[:, 1]
    pz = pos[:, 2]
    vx, vy, vz = _edge_vec_call(px, py, pz, src_flat)
    edge_vec = jnp.stack([vx, vy, vz], axis=-1)
    dst = jnp.broadcast_to(
        jnp.arange(n, dtype=jnp.int32)[:, None], (n, _K)).reshape(-1)
    edge_index = jnp.stack([src_flat, dst], axis=0)
    return edge_index, wgt2d.reshape(-1), edge_vec
